# Initial kernel scaffold; baseline (speedup 1.0000x reference)
import jax
import jax.numpy as jnp
from jax.experimental import pallas as pl


def kernel(flat, past_lengths, W, b):
    raise NotImplementedError("write your pallas kernel here")


# speedup vs baseline: 11.4323x; 11.4323x over previous
import jax
import jax.numpy as jnp
from jax.experimental import pallas as pl
from jax.experimental.pallas import tpu as pltpu

_ROWS = 2048


def _pool_kernel(len_ref, flat_ref, w_ref, bias_ref, out_ref, acc_ref):
    pid = pl.program_id(0)
    nsteps = pl.num_programs(0)

    @pl.when(pid == 0)
    def _init():
        acc_ref[...] = jnp.zeros_like(acc_ref)

    lengths = len_ref[...]
    nseg = lengths.shape[1]
    row_i = jax.lax.broadcasted_iota(jnp.int32, (nseg, nseg), 0)
    col_i = jax.lax.broadcasted_iota(jnp.int32, (nseg, nseg), 1)
    tri = jnp.where(row_i <= col_i, 1.0, 0.0)
    ends_f = jax.lax.dot_general(
        lengths.astype(jnp.float32), tri, (((1,), (0,)), ((), ())),
        preferred_element_type=jnp.float32,
    )
    ends = ends_f.astype(jnp.int32)
    starts = ends - lengths
    r0 = pid * _ROWS
    rows = r0 + jax.lax.broadcasted_iota(jnp.int32, (_ROWS, 1), 0)
    inv_len = 1.0 / jnp.maximum(lengths, 1).astype(jnp.float32)
    oh = jnp.where(jnp.logical_and(rows >= starts, rows < ends), inv_len, 0.0)
    tile = flat_ref[...]
    contrib = jax.lax.dot_general(
        oh, tile, (((0,), (0,)), ((), ())), preferred_element_type=jnp.float32
    )
    acc_ref[...] += contrib

    @pl.when(pid == nsteps - 1)
    def _finish():
        out_ref[...] = (
            jnp.dot(acc_ref[...], w_ref[...], preferred_element_type=jnp.float32)
            + bias_ref[...]
        )


def kernel(flat, past_lengths, W, b):
    total, d = flat.shape
    nseg = past_lengths.shape[0]
    lengths2d = past_lengths.reshape(1, nseg).astype(jnp.int32)
    bias2d = b.reshape(1, d)
    grid = total // _ROWS
    return pl.pallas_call(
        _pool_kernel,
        grid=(grid,),
        in_specs=[
            pl.BlockSpec((1, nseg), lambda i: (0, 0)),
            pl.BlockSpec((_ROWS, d), lambda i: (i, 0)),
            pl.BlockSpec((d, d), lambda i: (0, 0)),
            pl.BlockSpec((1, d), lambda i: (0, 0)),
        ],
        out_specs=pl.BlockSpec((nseg, d), lambda i: (0, 0)),
        out_shape=jax.ShapeDtypeStruct((nseg, d), jnp.float32),
        scratch_shapes=[pltpu.VMEM((nseg, d), jnp.float32)],
    )(lengths2d, flat, W, bias2d)
